# Initial kernel scaffold; baseline (speedup 1.0000x reference)
import jax
import jax.numpy as jnp
from jax.experimental import pallas as pl


def kernel(offsets, gt_labels, gt_nr_skel, gt_dt_norm, gt_dt_grad):
    raise NotImplementedError("write your pallas kernel here")


# speedup vs baseline: 12.3091x; 12.3091x over previous
import functools

import jax
import jax.numpy as jnp
from jax import lax
from jax.experimental import pallas as pl
from jax.experimental.pallas import tpu as pltpu

B, S, H, W = 4, 2, 512, 512
K = 8
DELTA_PUSH = 20.0
W_PULL, W_PUSH, W_PEN, W_BEN = 1.0, 1.0, 1.0, 5.0

BH = 64
R = H // BH


def _dense_body(off_x, off_y, nr_x, nr_y, gr_x, gr_y, lab,
                misc_out, sx_out, sy_out, cnt_out,
                acc_misc, acc_sx, acc_sy, acc_cnt):
    b = pl.program_id(0)
    r = pl.program_id(1)

    @pl.when(jnp.logical_and(b == 0, r == 0))
    def _init():
        acc_misc[...] = jnp.zeros_like(acc_misc)

    @pl.when(r == 0)
    def _init_batch():
        acc_sx[...] = jnp.zeros_like(acc_sx)
        acc_sy[...] = jnp.zeros_like(acc_sy)
        acc_cnt[...] = jnp.zeros_like(acc_cnt)

    ox = off_x[0, 0]
    oy = off_y[0, 0]
    nx = nr_x[0, 0]
    ny = nr_y[0, 0]
    gx_ = gr_x[0, 0]
    gy_ = gr_y[0, 0]
    labs = lab[0]

    col = lax.broadcasted_iota(jnp.int32, (BH, W), 1).astype(jnp.float32)
    row = (lax.broadcasted_iota(jnp.int32, (BH, W), 0) + r * BH).astype(jnp.float32)
    ex = col + ox
    ey = row + oy

    fg = (labs > 0).astype(jnp.float32)

    d2 = (ex - nx) ** 2 + (ey - ny) ** 2 + 1e-8
    dist = jnp.sqrt(d2)
    acc_misc[0:1, :] += jnp.sum(dist * fg, axis=0, keepdims=True)

    off_n = jnp.sqrt(ox * ox + oy * oy + 1e-12)
    grad_n = jnp.sqrt(gx_ * gx_ + gy_ * gy_ + 1e-12)
    cos = (ox * gx_ + oy * gy_) / (off_n * grad_n + 1e-8)
    acc_misc[1:2, :] += jnp.sum((1.0 - cos) * fg, axis=0, keepdims=True)

    acc_misc[2:3, :] += jnp.sum(fg, axis=0, keepdims=True)

    for k in range(K):
        m = (labs == k).astype(jnp.float32)
        acc_sx[k:k + 1, :] += jnp.sum(m * ex, axis=0, keepdims=True)
        acc_sy[k:k + 1, :] += jnp.sum(m * ey, axis=0, keepdims=True)
        acc_cnt[k:k + 1, :] += jnp.sum(m, axis=0, keepdims=True)

    @pl.when(r == R - 1)
    def _flush_batch():
        sx_out[0] = acc_sx[...]
        sy_out[0] = acc_sy[...]
        cnt_out[0] = acc_cnt[...]

    @pl.when(jnp.logical_and(b == B - 1, r == R - 1))
    def _flush():
        misc_out[...] = acc_misc[...]


def _dense_terms(offsets, gt_labels, gt_nr_skel, gt_dt_grad):
    grid = (B, R)
    blk4 = (1, 1, BH, W)

    def chan(c):
        return pl.BlockSpec(blk4, lambda b, r, c=c: (b, c, r, 0))

    lab_spec = pl.BlockSpec((1, BH, W), lambda b, r: (b, r, 0))

    out_shapes = [
        jax.ShapeDtypeStruct((8, W), jnp.float32),
        jax.ShapeDtypeStruct((B, K, W), jnp.float32),
        jax.ShapeDtypeStruct((B, K, W), jnp.float32),
        jax.ShapeDtypeStruct((B, K, W), jnp.float32),
    ]
    out_specs = [
        pl.BlockSpec((8, W), lambda b, r: (0, 0)),
        pl.BlockSpec((1, K, W), lambda b, r: (b, 0, 0)),
        pl.BlockSpec((1, K, W), lambda b, r: (b, 0, 0)),
        pl.BlockSpec((1, K, W), lambda b, r: (b, 0, 0)),
    ]
    return pl.pallas_call(
        _dense_body,
        grid=grid,
        in_specs=[chan(0), chan(1), chan(0), chan(1), chan(0), chan(1), lab_spec],
        out_specs=out_specs,
        out_shape=out_shapes,
        scratch_shapes=[
            pltpu.VMEM((8, W), jnp.float32),
            pltpu.VMEM((K, W), jnp.float32),
            pltpu.VMEM((K, W), jnp.float32),
            pltpu.VMEM((K, W), jnp.float32),
        ],
    )(offsets, offsets, gt_nr_skel, gt_nr_skel, gt_dt_grad, gt_dt_grad, gt_labels)


def _bilinear_jax(img, x, y):
    Hh, Ww = img.shape
    x = jnp.clip(x, 0.0, Ww - 1.0)
    y = jnp.clip(y, 0.0, Hh - 1.0)
    x0 = jnp.floor(x).astype(jnp.int32)
    y0 = jnp.floor(y).astype(jnp.int32)
    x1 = jnp.clip(x0 + 1, 0, Ww - 1)
    y1 = jnp.clip(y0 + 1, 0, Hh - 1)
    wx1 = x - x0.astype(jnp.float32)
    wx0 = 1.0 - wx1
    wy1 = y - y0.astype(jnp.float32)
    wy0 = 1.0 - wy1
    return (img[y0, x0] * wx0 * wy0 + img[y0, x1] * wx1 * wy0 +
            img[y1, x0] * wx0 * wy1 + img[y1, x1] * wx1 * wy1)


def kernel(offsets, gt_labels, gt_nr_skel, gt_dt_norm, gt_dt_grad):
    misc, sx, sy, cnt = _dense_terms(offsets, gt_labels, gt_nr_skel, gt_dt_grad)

    pull_sum = jnp.sum(misc[0])
    pen_sum = jnp.sum(misc[1])
    n_fg = jnp.maximum(jnp.sum(misc[2]), 1.0)

    sums_x = jnp.sum(sx, axis=-1)
    sums_y = jnp.sum(sy, axis=-1)
    cnts = jnp.sum(cnt, axis=-1)

    mu = jnp.stack([sums_x, sums_y], axis=-1) / jnp.maximum(cnts, 1.0)[..., None]
    valid = ((cnts > 0) & (jnp.arange(K)[None, :] > 0)).astype(jnp.float32)
    dmu = jnp.sqrt(jnp.sum((mu[:, :, None, :] - mu[:, None, :, :]) ** 2, axis=-1) + 1e-8)
    pm = valid[:, :, None] * valid[:, None, :] * (1.0 - jnp.eye(K)[None])
    hinge = jnp.maximum(DELTA_PUSH - dmu, 0.0) ** 2
    l_push = jnp.sum(hinge * pm) / jnp.maximum(jnp.sum(pm), 1.0)

    gy, gx = jnp.meshgrid(jnp.arange(H, dtype=jnp.float32),
                          jnp.arange(W, dtype=jnp.float32), indexing='ij')
    ex = gx[None] + offsets[:, 0]
    ey = gy[None] + offsets[:, 1]
    fg = (gt_labels > 0).astype(jnp.float32)
    sampled = jax.vmap(_bilinear_jax)(gt_dt_norm[:, 0], ex, ey)
    ben_sum = jnp.sum((1.0 - sampled) * fg)

    l_pull = pull_sum / n_fg
    l_pen = pen_sum / n_fg
    l_ben = ben_sum / n_fg
    total = W_PULL * l_pull + W_PUSH * l_push + W_PEN * l_pen + W_BEN * l_ben
    return total, l_pull, l_push, l_pen, l_ben
